# baseline (speedup 1.0000x reference)
import functools

import jax
import jax.numpy as jnp
from jax import lax
from jax.experimental import pallas as pl
from jax.experimental.pallas import tpu as pltpu
from jax.experimental.pallas import tpu_sc as plsc

BATCH = 16384
FIELDS = 26
EMBED = 32
NC = 2
NS = 16
NW = NC * NS
ROWS_W = BATCH // NW
PADF = 32
CHUNK = 32
NCHUNK = ROWS_W // CHUNK
NBUF = 3
LANES = 16


def _emb_body(idx_hbm, table_hbm, out_hbm, idx_v, flat_v, rows_v, *sems):
    gsems = sems[:NBUF]
    osems = sems[NBUF:]
    wid = lax.axis_index("s") * NC + lax.axis_index("c")
    base = wid * ROWS_W

    pltpu.sync_copy(idx_hbm.at[pl.ds(base, ROWS_W), :], idx_v)

    lane = lax.iota(jnp.int32, LANES)
    chi = jnp.minimum(lane + LANES, FIELDS - 1)

    def repack_step(r, carry):
        rv = lane * 0 + r
        flat_v[pl.ds(r * PADF, LANES)] = plsc.load_gather(idx_v, [rv, lane])
        flat_v[pl.ds(r * PADF + LANES, LANES)] = plsc.load_gather(
            idx_v, [rv, chi])
        return carry

    lax.fori_loop(0, ROWS_W, repack_step, 0)

    def start_chunk(g):
        b = g % NBUF

        def row_gather(k, carry):
            pltpu.async_copy(
                table_hbm.at[flat_v.at[pl.ds((g * CHUNK + k) * PADF, FIELDS)]],
                rows_v.at[b].at[k], gsems[b])
            return carry

        lax.fori_loop(0, CHUNK, row_gather, 0)

    def drain_chunk(g):
        b = g % NBUF
        pltpu.make_async_copy(
            out_hbm.at[pl.ds(0, CHUNK)], rows_v.at[b], gsems[b]).wait()

    ods = [None] * NCHUNK
    for g in range(min(NBUF, NCHUNK)):
        start_chunk(g)
    for g in range(NCHUNK):
        b = g % NBUF
        drain_chunk(g)
        ods[g] = pltpu.async_copy(
            rows_v.at[b],
            out_hbm.at[pl.ds(base + g * CHUNK, CHUNK)],
            osems[b])
        nxt = g + NBUF
        if nxt < NCHUNK:
            ods[g].wait()
            start_chunk(nxt)
    for g in range(max(NCHUNK - NBUF, 0), NCHUNK):
        ods[g].wait()


@jax.jit
def kernel(x, emb_weight):
    idx = x.astype(jnp.int32)
    mesh = plsc.VectorSubcoreMesh(core_axis_name="c", subcore_axis_name="s")
    run = functools.partial(
        pl.kernel,
        out_type=jax.ShapeDtypeStruct((BATCH, FIELDS, EMBED), jnp.float32),
        mesh=mesh,
        scratch_types=[
            pltpu.VMEM((ROWS_W, FIELDS), jnp.int32),
            pltpu.VMEM((ROWS_W * PADF,), jnp.int32),
            pltpu.VMEM((NBUF, CHUNK, FIELDS, EMBED), jnp.float32),
        ] + [pltpu.SemaphoreType.DMA] * (2 * NBUF),
        compiler_params=pltpu.CompilerParams(
            use_tc_tiling_on_sc=False, needs_layout_passes=False),
    )(_emb_body)
    return run(idx, emb_weight)
